# baseline (speedup 1.0000x reference)
import functools
import jax
import jax.numpy as jnp
from jax import lax
from jax.experimental import pallas as pl
from jax.experimental.pallas import tpu as pltpu
from jax.experimental.pallas import tpu_sc as plsc

N = 10000
E = 320000
D = 128
DW = 144
B = 2

NTILES = 16
EP = E // NTILES
K = 40
G = 5
NGRP = EP // (G * K)
SB = 2 * G
NSB = NGRP // 2
R = 5
DP = 3
ROWS_PER_TILE = 640
NP = NTILES * ROWS_PER_TILE


def _sc_body(xa, srce, dste, zrow,
             agg_out,
             agg_sh, svA, dvA, svB, dvB,
             rows0, rows1, rows2, rows3, rows4,
             g0, g1, g2s, g3, g4, s0, s1, s2, s3, s4, isemA, isemB):
    c = lax.axis_index("c")
    s = lax.axis_index("s")
    rbase = s * ROWS_PER_TILE
    rows = (rows0, rows1, rows2, rows3, rows4)
    gsem = (g0, g1, g2s, g3, g4)
    ssem = (s0, s1, s2, s3, s4)
    sv = (svA, svB)
    dv = (dvA, dvB)
    isem = (isemA, isemB)

    pltpu.sync_copy(zrow.at[pl.ds(rbase, ROWS_PER_TILE)],
                    agg_sh.at[pl.ds(rbase, ROWS_PER_TILE)])
    plsc.subcore_barrier()

    def gather(p, r, b):
        pltpu.async_copy(xa.at[sv[p].at[r]], rows[b], gsem[b])

    def gwait(b):
        pltpu.make_async_copy(xa.at[pl.ds(0, K)], rows[b], gsem[b]).wait()

    def swait(b):
        pltpu.make_async_copy(xa.at[pl.ds(0, K)], rows[b], ssem[b]).wait()

    def stage(p, g):
        pltpu.async_copy(srce.at[c, s, g], sv[p], isem[p])
        pltpu.async_copy(dste.at[s, g], dv[p], isem[p])

    def stage_wait(p):
        pltpu.make_async_copy(srce.at[c, s, 0], sv[p], isem[p]).wait()
        pltpu.make_async_copy(dste.at[s, 0], dv[p], isem[p]).wait()

    pltpu.sync_copy(srce.at[c, s, 0], svA)
    pltpu.sync_copy(dste.at[s, 0], dvA)
    for j in range(DP):
        gather(0, j, j)

    def sblock(g2, carry):
        for j in range(SB):
            b = j % R
            p, r = (0, j) if j < G else (1, j - G)
            gwait(b)
            jn = j + DP
            bn = jn % R
            if j < R - DP:
                @pl.when(g2 > 0)
                def _():
                    swait(bn)
            else:
                swait(bn)
            if j == 1:
                stage(1, 2 * g2 + 1)
            if j == 6:
                stage(0, jnp.minimum(2 * g2 + 2, NGRP - 1))
            if jn == G:
                stage_wait(1)
            if jn == SB:
                stage_wait(0)
            pn, rn = (0, jn) if jn < G else (1, jn - G)
            if jn >= SB:
                pn, rn = 0, jn - SB
            gather(pn, rn, bn)
            pltpu.async_copy(rows[b], agg_sh.at[dv[p].at[r]], ssem[b],
                             add=True)
        return carry

    lax.fori_loop(0, NSB, sblock, 0)
    for j in range(DP):
        gwait(j % R)
    swait((SB - 2) % R)
    swait((SB - 1) % R)
    plsc.subcore_barrier()

    @pl.when(s < NTILES - 1)
    def _():
        pltpu.sync_copy(agg_sh.at[pl.ds(rbase, ROWS_PER_TILE)],
                        agg_out.at[pl.ds(c * N + rbase, ROWS_PER_TILE)])

    @pl.when(s == NTILES - 1)
    def _():
        last = N - (NTILES - 1) * ROWS_PER_TILE
        base = (NTILES - 1) * ROWS_PER_TILE
        pltpu.sync_copy(agg_sh.at[pl.ds(base, last)],
                        agg_out.at[pl.ds(c * N + base, last)])


_sc_agg = functools.partial(
    pl.kernel,
    out_type=jax.ShapeDtypeStruct((B * N, DW), jnp.float32),
    mesh=plsc.VectorSubcoreMesh(core_axis_name="c", subcore_axis_name="s"),
    scratch_types=[
        pltpu.VMEM_SHARED((NP, DW), jnp.float32),
        pltpu.VMEM((G, K), jnp.int32),
        pltpu.VMEM((G, K), jnp.int32),
        pltpu.VMEM((G, K), jnp.int32),
        pltpu.VMEM((G, K), jnp.int32),
        pltpu.VMEM((K, DW), jnp.float32),
        pltpu.VMEM((K, DW), jnp.float32),
        pltpu.VMEM((K, DW), jnp.float32),
        pltpu.VMEM((K, DW), jnp.float32),
        pltpu.VMEM((K, DW), jnp.float32),
        pltpu.SemaphoreType.DMA,
        pltpu.SemaphoreType.DMA,
        pltpu.SemaphoreType.DMA,
        pltpu.SemaphoreType.DMA,
        pltpu.SemaphoreType.DMA,
        pltpu.SemaphoreType.DMA,
        pltpu.SemaphoreType.DMA,
        pltpu.SemaphoreType.DMA,
        pltpu.SemaphoreType.DMA,
        pltpu.SemaphoreType.DMA,
        pltpu.SemaphoreType.DMA,
        pltpu.SemaphoreType.DMA,
    ],
    compiler_params=pltpu.CompilerParams(use_tc_tiling_on_sc=False),
)(_sc_body)


def _tc_body(agg_ref, x_ref, wl_ref, bl_ref, wr_ref, out_ref):
    deg = jnp.maximum(agg_ref[:, D:D + 1], 1.0)
    aggn = agg_ref[:, :D] / deg
    out_ref[...] = (
        jnp.dot(aggn, wl_ref[...], preferred_element_type=jnp.float32)
        + bl_ref[...]
        + jnp.dot(x_ref[...], wr_ref[...], preferred_element_type=jnp.float32)
    )


RB = 1000


def _tc_epilogue(agg, xf, wlT, bl, wrT):
    nb = (B * N) // RB
    return pl.pallas_call(
        _tc_body,
        grid=(nb,),
        in_specs=[
            pl.BlockSpec((RB, DW), lambda i: (i, 0)),
            pl.BlockSpec((RB, D), lambda i: (i, 0)),
            pl.BlockSpec((D, D), lambda i: (0, 0)),
            pl.BlockSpec((1, D), lambda i: (0, 0)),
            pl.BlockSpec((D, D), lambda i: (0, 0)),
        ],
        out_specs=pl.BlockSpec((RB, D), lambda i: (i, 0)),
        out_shape=jax.ShapeDtypeStruct((B * N, D), jnp.float32),
    )(agg, xf, wlT, bl, wrT)


@jax.jit
def kernel(x, edge_index, W_l, b_l, W_r):
    src = edge_index[0].reshape(NTILES, NGRP, G, K)
    dste = edge_index[1].reshape(NTILES, NGRP, G, K)
    srce = jnp.stack([src, src + N])
    xf = x.reshape(B * N, D)
    xa = jnp.concatenate([xf, jnp.ones((B * N, DW - D), jnp.float32)], axis=1)
    zrow = jnp.zeros((NP, DW), jnp.float32)
    agg = _sc_agg(xa, srce, dste, zrow)
    out = _tc_epilogue(agg, xf, W_l.T, b_l.reshape(1, D), W_r.T)
    return out.reshape(B, N, D)


# speedup vs baseline: 1.3218x; 1.3218x over previous
import functools
import jax
import jax.numpy as jnp
from jax import lax
from jax.experimental import pallas as pl
from jax.experimental.pallas import tpu as pltpu
from jax.experimental.pallas import tpu_sc as plsc

N = 10000
E = 320000
D = 128
B = 2

NTILES = 16
EP = E // NTILES
K = 40
NCH = EP // K
SB = 10
NSB = NCH // SB
R = 5
DP = 3
SL = 7
ROWS_PER_TILE = 640
NP = NTILES * ROWS_PER_TILE
DIVBLK = ROWS_PER_TILE // K


def _sc_body(xf, sde, z128, z16, ones_h,
             agg_out,
             agg_sh, deg_sh,
             i0, i1, i2, i3, i4, i5, i6, i7, i8, i9,
             d0, d1, d2, d3, d4, d5, d6, d7, d8, d9,
             rows0, rows1, rows2, rows3, rows4, ones_v,
             g0, g1, g2s, g3, g4, s0, s1, s2, s3, s4, isem, dsem):
    c = lax.axis_index("c")
    s = lax.axis_index("s")
    rbase = s * ROWS_PER_TILE
    tb = s * EP
    rows = (rows0, rows1, rows2, rows3, rows4)
    gsem = (g0, g1, g2s, g3, g4)
    ssem = (s0, s1, s2, s3, s4)
    siv = (i0, i1, i2, i3, i4, i5, i6, i7, i8, i9)
    div = (d0, d1, d2, d3, d4, d5, d6, d7, d8, d9)

    pltpu.sync_copy(z128.at[pl.ds(rbase, ROWS_PER_TILE)],
                    agg_sh.at[pl.ds(rbase, ROWS_PER_TILE)])
    pltpu.sync_copy(z16.at[pl.ds(rbase, ROWS_PER_TILE)],
                    deg_sh.at[pl.ds(rbase, ROWS_PER_TILE)])
    pltpu.sync_copy(ones_h, ones_v)
    plsc.subcore_barrier()

    def stage(m, pair):
        off = tb + jnp.minimum(m, NCH - 1) * K
        pltpu.async_copy(sde.at[c, pl.ds(off, K)], siv[pair], isem)
        pltpu.async_copy(sde.at[2, pl.ds(off, K)], div[pair], isem)

    def stage_wait():
        pltpu.make_async_copy(sde.at[0, pl.ds(0, K)], siv[0], isem).wait()
        pltpu.make_async_copy(sde.at[0, pl.ds(0, K)], div[0], isem).wait()

    def gather(pair, b):
        pltpu.async_copy(xf.at[siv[pair]], rows[b], gsem[b])

    def gwait(b):
        pltpu.make_async_copy(xf.at[pl.ds(0, K)], rows[b], gsem[b]).wait()

    def swait(b):
        pltpu.make_async_copy(xf.at[pl.ds(0, K)], rows[b], ssem[b]).wait()

    def dwait():
        pltpu.make_async_copy(ones_h, ones_v, dsem).wait()

    for m in range(SL):
        stage(m, m)
    for j in range(DP):
        stage_wait()
        gather(j, j)

    def sblock(g2, carry):
        for j in range(SB):
            b = j % R
            bn = (j + DP) % R
            gwait(b)
            if j < R - DP:
                @pl.when(g2 > 0)
                def _():
                    swait(bn)
                    dwait()
            else:
                swait(bn)
                dwait()
            stage_wait()
            gather((j + DP) % SB, bn)
            stage(g2 * SB + j + SL, (j + SL) % SB)
            pltpu.async_copy(rows[b], agg_sh.at[div[j]], ssem[b], add=True)
            pltpu.async_copy(ones_v, deg_sh.at[div[j]], dsem, add=True)
        return carry

    lax.fori_loop(0, NSB, sblock, 0)
    for j in range(DP):
        gwait(j % R)
    swait((SB - 2) % R)
    swait((SB - 1) % R)
    dwait()
    dwait()
    for _ in range(SL - DP):
        stage_wait()
    plsc.subcore_barrier()

    def divide_out(nblk):
        def blk(i, carry):
            rb = rbase + i * K
            pltpu.sync_copy(agg_sh.at[pl.ds(rb, K)], rows0)
            pltpu.sync_copy(deg_sh.at[pl.ds(rb, K)], ones_v)

            def row(r, carry2):
                rv = 1.0 / jnp.maximum(ones_v[r], 1.0)
                for k in range(D // 16):
                    sl = pl.ds(k * 16, 16)
                    rows0[r, sl] = rows0[r, sl] * rv
                return carry2

            lax.fori_loop(0, K, row, 0)
            pltpu.sync_copy(rows0, agg_out.at[pl.ds(c * N + rb, K)])
            return carry

        lax.fori_loop(0, nblk, blk, 0)

    @pl.when(s < NTILES - 1)
    def _():
        divide_out(DIVBLK)

    @pl.when(s == NTILES - 1)
    def _():
        divide_out((N - (NTILES - 1) * ROWS_PER_TILE) // K)


_sc_agg = functools.partial(
    pl.kernel,
    out_type=jax.ShapeDtypeStruct((B * N, D), jnp.float32),
    mesh=plsc.VectorSubcoreMesh(core_axis_name="c", subcore_axis_name="s"),
    scratch_types=(
        [pltpu.VMEM_SHARED((NP, D), jnp.float32),
         pltpu.VMEM_SHARED((NP, 16), jnp.float32)]
        + [pltpu.VMEM((K,), jnp.int32) for _ in range(2 * SB)]
        + [pltpu.VMEM((K, D), jnp.float32) for _ in range(R)]
        + [pltpu.VMEM((K, 16), jnp.float32)]
        + [pltpu.SemaphoreType.DMA for _ in range(2 * R + 2)]
    ),
    compiler_params=pltpu.CompilerParams(use_tc_tiling_on_sc=False),
)(_sc_body)


def _tc_body(agg_ref, x_ref, wl_ref, bl_ref, wr_ref, out_ref):
    out_ref[...] = (
        jnp.dot(agg_ref[...], wl_ref[...], preferred_element_type=jnp.float32)
        + bl_ref[...]
        + jnp.dot(x_ref[...], wr_ref[...], preferred_element_type=jnp.float32)
    )


RB = 1000


def _tc_epilogue(agg, xf, wlT, bl, wrT):
    nb = (B * N) // RB
    return pl.pallas_call(
        _tc_body,
        grid=(nb,),
        in_specs=[
            pl.BlockSpec((RB, D), lambda i: (i, 0)),
            pl.BlockSpec((RB, D), lambda i: (i, 0)),
            pl.BlockSpec((D, D), lambda i: (0, 0)),
            pl.BlockSpec((1, D), lambda i: (0, 0)),
            pl.BlockSpec((D, D), lambda i: (0, 0)),
        ],
        out_specs=pl.BlockSpec((RB, D), lambda i: (i, 0)),
        out_shape=jax.ShapeDtypeStruct((B * N, D), jnp.float32),
    )(agg, xf, wlT, bl, wrT)


@jax.jit
def kernel(x, edge_index, W_l, b_l, W_r):
    src = edge_index[0]
    dst = edge_index[1]
    sde = jnp.stack([src, src + N, dst])
    xf = x.reshape(B * N, D)
    z128 = jnp.zeros((NP, D), jnp.float32)
    z16 = jnp.zeros((NP, 16), jnp.float32)
    ones_h = jnp.ones((K, 16), jnp.float32)
    agg = _sc_agg(xf, sde, z128, z16, ones_h)
    out = _tc_epilogue(agg, xf, W_l.T, b_l.reshape(1, D), W_r.T)
    return out.reshape(B, N, D)


# speedup vs baseline: 1.4555x; 1.1011x over previous
import functools
import jax
import jax.numpy as jnp
from jax import lax
from jax.experimental import pallas as pl
from jax.experimental.pallas import tpu as pltpu
from jax.experimental.pallas import tpu_sc as plsc

N = 10000
E = 320000
D = 128
B = 2

NTILES = 16
EP = E // NTILES
K = 40
NCH = EP // K
SB = 10
NSB = NCH // SB
R = 5
DP = 4
SL = 7
ROWS_PER_TILE = 640
NP = NTILES * ROWS_PER_TILE
DIVBLK = ROWS_PER_TILE // K


def _sc_body(xf, sde, z128, z16, ones_h,
             agg_out,
             agg_sh, deg_sh,
             i0, i1, i2, i3, i4, i5, i6, i7, i8, i9,
             d0, d1, d2, d3, d4, d5, d6, d7, d8, d9,
             rows0, rows1, rows2, rows3, rows4, ones_v,
             g0, g1, g2s, g3, g4, s0, s1, s2, s3, s4, isem, dsem):
    c = lax.axis_index("c")
    s = lax.axis_index("s")
    rbase = s * ROWS_PER_TILE
    tb = s * EP
    rows = (rows0, rows1, rows2, rows3, rows4)
    gsem = (g0, g1, g2s, g3, g4)
    ssem = (s0, s1, s2, s3, s4)
    siv = (i0, i1, i2, i3, i4, i5, i6, i7, i8, i9)
    div = (d0, d1, d2, d3, d4, d5, d6, d7, d8, d9)

    pltpu.sync_copy(z128.at[pl.ds(rbase, ROWS_PER_TILE)],
                    agg_sh.at[pl.ds(rbase, ROWS_PER_TILE)])
    pltpu.sync_copy(z16.at[pl.ds(rbase, ROWS_PER_TILE)],
                    deg_sh.at[pl.ds(rbase, ROWS_PER_TILE)])
    pltpu.sync_copy(ones_h, ones_v)
    plsc.subcore_barrier()

    def stage(m, pair):
        off = tb + jnp.minimum(m, NCH - 1) * K
        pltpu.async_copy(sde.at[c, pl.ds(off, K)], siv[pair], isem)
        pltpu.async_copy(sde.at[2, pl.ds(off, K)], div[pair], isem)

    def stage_wait():
        pltpu.make_async_copy(sde.at[0, pl.ds(0, K)], siv[0], isem).wait()
        pltpu.make_async_copy(sde.at[0, pl.ds(0, K)], div[0], isem).wait()

    def gather(pair, b):
        pltpu.async_copy(xf.at[siv[pair]], rows[b], gsem[b])

    def gwait(b):
        pltpu.make_async_copy(xf.at[pl.ds(0, K)], rows[b], gsem[b]).wait()

    def swait(b):
        pltpu.make_async_copy(xf.at[pl.ds(0, K)], rows[b], ssem[b]).wait()

    def dwait():
        pltpu.make_async_copy(ones_h, ones_v, dsem).wait()

    for m in range(SL):
        stage(m, m)
    for j in range(DP):
        stage_wait()
        gather(j, j)

    def sblock(g2, carry):
        for j in range(SB):
            b = j % R
            bn = (j + DP) % R
            gwait(b)
            if j < R - DP:
                @pl.when(g2 > 0)
                def _():
                    swait(bn)
                    dwait()
            else:
                swait(bn)
                dwait()
            stage_wait()
            gather((j + DP) % SB, bn)
            stage(g2 * SB + j + SL, (j + SL) % SB)
            pltpu.async_copy(rows[b], agg_sh.at[div[j]], ssem[b], add=True)
            pltpu.async_copy(ones_v, deg_sh.at[div[j]], dsem, add=True)
        return carry

    lax.fori_loop(0, NSB, sblock, 0)
    for j in range(DP):
        gwait(j % R)
    for j in range(R - DP):
        swait((SB - (R - DP) + j) % R)
        dwait()
    for _ in range(SL - DP):
        stage_wait()
    plsc.subcore_barrier()

    def divide_out(nblk):
        def blk(i, carry):
            rb = rbase + i * K
            pltpu.sync_copy(agg_sh.at[pl.ds(rb, K)], rows0)
            pltpu.sync_copy(deg_sh.at[pl.ds(rb, K)], ones_v)

            def row(r, carry2):
                rv = 1.0 / jnp.maximum(ones_v[r], 1.0)
                for k in range(D // 16):
                    sl = pl.ds(k * 16, 16)
                    rows0[r, sl] = rows0[r, sl] * rv
                return carry2

            lax.fori_loop(0, K, row, 0)
            pltpu.sync_copy(rows0, agg_out.at[pl.ds(c * N + rb, K)])
            return carry

        lax.fori_loop(0, nblk, blk, 0)

    @pl.when(s < NTILES - 1)
    def _():
        divide_out(DIVBLK)

    @pl.when(s == NTILES - 1)
    def _():
        divide_out((N - (NTILES - 1) * ROWS_PER_TILE) // K)


_sc_agg = functools.partial(
    pl.kernel,
    out_type=jax.ShapeDtypeStruct((B * N, D), jnp.float32),
    mesh=plsc.VectorSubcoreMesh(core_axis_name="c", subcore_axis_name="s"),
    scratch_types=(
        [pltpu.VMEM_SHARED((NP, D), jnp.float32),
         pltpu.VMEM_SHARED((NP, 16), jnp.float32)]
        + [pltpu.VMEM((K,), jnp.int32) for _ in range(2 * SB)]
        + [pltpu.VMEM((K, D), jnp.float32) for _ in range(R)]
        + [pltpu.VMEM((K, 16), jnp.float32)]
        + [pltpu.SemaphoreType.DMA for _ in range(2 * R + 2)]
    ),
    compiler_params=pltpu.CompilerParams(use_tc_tiling_on_sc=False),
)(_sc_body)


def _tc_body(agg_ref, x_ref, wl_ref, bl_ref, wr_ref, out_ref):
    out_ref[...] = (
        jnp.dot(agg_ref[...], wl_ref[...], preferred_element_type=jnp.float32)
        + bl_ref[...]
        + jnp.dot(x_ref[...], wr_ref[...], preferred_element_type=jnp.float32)
    )


RB = 1000


def _tc_epilogue(agg, xf, wlT, bl, wrT):
    nb = (B * N) // RB
    return pl.pallas_call(
        _tc_body,
        grid=(nb,),
        in_specs=[
            pl.BlockSpec((RB, D), lambda i: (i, 0)),
            pl.BlockSpec((RB, D), lambda i: (i, 0)),
            pl.BlockSpec((D, D), lambda i: (0, 0)),
            pl.BlockSpec((1, D), lambda i: (0, 0)),
            pl.BlockSpec((D, D), lambda i: (0, 0)),
        ],
        out_specs=pl.BlockSpec((RB, D), lambda i: (i, 0)),
        out_shape=jax.ShapeDtypeStruct((B * N, D), jnp.float32),
    )(agg, xf, wlT, bl, wrT)


@jax.jit
def kernel(x, edge_index, W_l, b_l, W_r):
    src = edge_index[0]
    dst = edge_index[1]
    sde = jnp.stack([src, src + N, dst])
    xf = x.reshape(B * N, D)
    z128 = jnp.zeros((NP, D), jnp.float32)
    z16 = jnp.zeros((NP, 16), jnp.float32)
    ones_h = jnp.ones((K, 16), jnp.float32)
    agg = _sc_agg(xf, sde, z128, z16, ones_h)
    out = _tc_epilogue(agg, xf, W_l.T, b_l.reshape(1, D), W_r.T)
    return out.reshape(B, N, D)
